# baseline (speedup 1.0000x reference)
import jax
import jax.numpy as jnp
from jax import lax
from jax.experimental import pallas as pl
from jax.experimental.pallas import tpu as pltpu
from jax.experimental.pallas import tpu_sc as plsc

N = 10000
E = 320000
HID = 128
OUT = 64
L = 8

NC = 2
NS = 16
NW = NC * NS
EPT = E // NW
B = 80
NCHUNK = EPT // B
GROUPS = B // 16
RB = 125
NEG = -1.0e30

_INV_SQRT_HID = 1.0 / (HID ** 0.5)


def _mesh():
    return plsc.VectorSubcoreMesh(
        core_axis_name="c", subcore_axis_name="s", num_cores=NC, num_subcores=NS
    )


_SC_PARAMS = pltpu.CompilerParams(needs_layout_passes=False)


def _sc_pass_a(q_hbm, k_hbm, src_hbm, dst_hbm, alpha_out, mloc_out,
               srcs, dsts, alpha_all, mloc, qr0, kr0, qr1, kr1,
               isem, qs0, ks0, qs1, ks1):
    wid = lax.axis_index("s") * NC + lax.axis_index("c")
    base = wid * EPT
    iota = lax.iota(jnp.int32, 16)

    ci1 = pltpu.async_copy(src_hbm.at[pl.ds(base, EPT)], srcs, isem)
    ci2 = pltpu.async_copy(dst_hbm.at[pl.ds(base, EPT)], dsts, isem)

    def init_body(i, _):
        mloc[pl.ds(i * 16, 16)] = jnp.full((16,), NEG, jnp.float32)
        return 0

    lax.fori_loop(0, N // 16, init_body, 0)
    ci1.wait()
    ci2.wait()

    def issue(c, qr, kr, qsem, ksem):
        idx = dsts.at[pl.ds(c * B, B)]
        idxs = srcs.at[pl.ds(c * B, B)]
        cq = pltpu.async_copy(q_hbm.at[idx], qr, qsem)
        ck = pltpu.async_copy(k_hbm.at[idxs], kr, ksem)
        return cq, ck

    def wait(qr, kr, qsem, ksem):
        pltpu.make_async_copy(q_hbm.at[dsts.at[pl.ds(0, B)]], qr, qsem).wait()
        pltpu.make_async_copy(k_hbm.at[srcs.at[pl.ds(0, B)]], kr, ksem).wait()

    def compute(c, qr, kr):
        off = c * B

        def group_body(g):
            al = jnp.zeros((16,), jnp.float32)
            for e in range(16):
                row = g * 16 + e
                p0 = jnp.zeros((16,), jnp.float32)
                p1 = jnp.zeros((16,), jnp.float32)
                for c8 in range(HID // 16):
                    qv = qr[row, pl.ds(c8 * 16, 16)]
                    kv = kr[row, pl.ds(c8 * 16, 16)]
                    if c8 % 2 == 0:
                        p0 = p0 + qv * kv
                    else:
                        p1 = p1 + qv * kv
                b = p0 + p1
                for st in (1, 2, 4, 8):
                    b = b + b.at[iota ^ st].get(mode="promise_in_bounds")
                al = jnp.where(iota == e, b, al)
            al = al * _INV_SQRT_HID
            alpha_all[pl.ds(off + g * 16, 16)] = al
            dstv = dsts[pl.ds(off + g * 16, 16)]
            ka, da = plsc.sort_key_val(al, dstv)
            old = plsc.load_gather(mloc, [da])
            plsc.store_scatter(mloc, [da], jnp.maximum(old, ka))
            kd, dd = plsc.sort_key_val(al, dstv, descending=True)
            old2 = plsc.load_gather(mloc, [dd])
            plsc.store_scatter(mloc, [dd], jnp.maximum(old2, kd))

        for g in range(GROUPS):
            group_body(g)

    i0 = issue(0, qr0, kr0, qs0, ks0)

    def pair_body(tt, _):
        c0 = 2 * tt
        c1 = c0 + 1

        @pl.when(c1 < NCHUNK)
        def _():
            issue(c1, qr1, kr1, qs1, ks1)

        wait(qr0, kr0, qs0, ks0)
        compute(c0, qr0, kr0)

        @pl.when(c1 < NCHUNK)
        def _():
            @pl.when(c1 + 1 < NCHUNK)
            def _():
                issue(c1 + 1, qr0, kr0, qs0, ks0)

            wait(qr1, kr1, qs1, ks1)
            compute(c1, qr1, kr1)

        return 0

    lax.fori_loop(0, (NCHUNK + 1) // 2, pair_body, 0)

    pltpu.sync_copy(alpha_all, alpha_out.at[pl.ds(base, EPT)])
    pltpu.sync_copy(mloc, mloc_out.at[pl.ds(wid * N, N)])


def _run_pass_a(q, k, src, dst):
    f = pl.kernel(
        _sc_pass_a,
        out_type=(
            jax.ShapeDtypeStruct((E,), jnp.float32),
            jax.ShapeDtypeStruct((NW * N,), jnp.float32),
        ),
        mesh=_mesh(),
        scratch_types=[
            pltpu.VMEM((EPT,), jnp.int32),
            pltpu.VMEM((EPT,), jnp.int32),
            pltpu.VMEM((EPT,), jnp.float32),
            pltpu.VMEM((N,), jnp.float32),
            pltpu.VMEM((B, HID), jnp.float32),
            pltpu.VMEM((B, HID), jnp.float32),
            pltpu.VMEM((B, HID), jnp.float32),
            pltpu.VMEM((B, HID), jnp.float32),
            pltpu.SemaphoreType.DMA,
            pltpu.SemaphoreType.DMA,
            pltpu.SemaphoreType.DMA,
            pltpu.SemaphoreType.DMA,
            pltpu.SemaphoreType.DMA,
        ],
        compiler_params=_SC_PARAMS,
    )
    return f(q, k, src, dst)


def _sc_pass_b(v_hbm, m_hbm, alpha_hbm, src_hbm, dst_hbm, agg_out, s_out,
               m_vm, s_loc, vrs, idxs, idxd, albs, aggsp,
               msem, isems, vsems, ssems):
    cid = lax.axis_index("c")
    sid = lax.axis_index("s")
    wid = sid * NC + cid
    base = wid * EPT
    iota = lax.iota(jnp.int32, 16)

    cm = pltpu.async_copy(m_hbm, m_vm, msem)

    def zinit_body(i, _):
        s_loc[pl.ds(i * 16, 16)] = jnp.zeros((16,), jnp.float32)
        return 0

    lax.fori_loop(0, N // 16, zinit_body, 0)

    def zrow_body(r, _):
        for cc in range(HID // 16):
            vrs[0][r, pl.ds(cc * 16, 16)] = jnp.zeros((16,), jnp.float32)
        return 0

    lax.fori_loop(0, B, zrow_body, 0)

    for t in range(RB // NS + 1):
        bi = t * NS + sid

        @pl.when(bi < RB)
        def _():
            off = pl.multiple_of(bi * B, B)
            pltpu.sync_copy(vrs[0], aggsp.at[pl.ds(off, B)])

    cm.wait()
    plsc.subcore_barrier()

    def issue_idx(c, j):
        off = c * B
        pltpu.async_copy(src_hbm.at[pl.ds(off + base, B)], idxs[j], isems[j])
        pltpu.async_copy(dst_hbm.at[pl.ds(off + base, B)], idxd[j], isems[j])
        pltpu.async_copy(alpha_hbm.at[pl.ds(off + base, B)], albs[j], isems[j])

    def wait_idx(j):
        pltpu.make_async_copy(src_hbm.at[pl.ds(0, B)], idxs[j], isems[j]).wait()
        pltpu.make_async_copy(dst_hbm.at[pl.ds(0, B)], idxd[j], isems[j]).wait()
        pltpu.make_async_copy(alpha_hbm.at[pl.ds(0, B)], albs[j], isems[j]).wait()

    def issue_v(j, p):
        pltpu.async_copy(v_hbm.at[idxs[j]], vrs[p], vsems[p])

    def wait_v(p):
        pltpu.make_async_copy(v_hbm.at[pl.ds(0, B)], vrs[p], vsems[p]).wait()

    def issue_scat(j, p):
        pltpu.async_copy(vrs[p], aggsp.at[idxd[j]], ssems[p], add=True)

    def wait_scat(p):
        pltpu.make_async_copy(vrs[p], aggsp.at[pl.ds(0, B)], ssems[p]).wait()

    def compute(j, p):
        vr = vrs[p]

        def group_body(g, _):
            al = albs[j][pl.ds(g * 16, 16)]
            dstv = idxd[j][pl.ds(g * 16, 16)]
            mg = plsc.load_gather(m_vm, [dstv])
            t = jnp.exp(al - mg)
            plsc.addupdate_scatter(s_loc, [dstv], t)
            for e in range(16):
                row = g * 16 + e
                te = t.at[jnp.full((16,), e, jnp.int32)].get(mode="promise_in_bounds")
                for c8 in range(HID // 16):
                    sl = pl.ds(c8 * 16, 16)
                    vr[row, sl] = vr[row, sl] * te
            return 0

        lax.fori_loop(0, GROUPS, group_body, 0)

    issue_idx(0, 0)
    issue_idx(1, 1)
    issue_idx(2, 2)
    wait_idx(0)
    issue_v(0, 0)

    def chunk_step(c, u):
        j = u % 3
        p = u % 2
        j1 = (u + 1) % 3
        j2 = (u + 2) % 3
        p1 = (u + 1) % 2

        @pl.when(c + 1 < NCHUNK)
        def _():
            wait_idx(j1)

        @pl.when(c >= 1)
        def _():
            wait_scat(p1)

        @pl.when(c + 1 < NCHUNK)
        def _():
            issue_v(j1, p1)

        @pl.when((c + 2 < NCHUNK) & (c >= 1))
        def _():
            issue_idx(c + 2, j2)

        wait_v(p)
        compute(j, p)
        issue_scat(j, p)

    def hex_body(h, _):
        for u in range(6):
            c = 6 * h + u

            @pl.when(c < NCHUNK)
            def _():
                chunk_step(c, u)

        return 0

    lax.fori_loop(0, (NCHUNK + 5) // 6, hex_body, 0)

    wait_scat((NCHUNK - 1) % 2)
    plsc.subcore_barrier()

    for t in range(RB // NS + 1):
        bi = t * NS + sid

        @pl.when(bi < RB)
        def _():
            off = pl.multiple_of(bi * B, B)
            pltpu.sync_copy(aggsp.at[pl.ds(off, B)],
                            agg_out.at[cid, pl.ds(off, B)])

    pltpu.sync_copy(s_loc, s_out.at[pl.ds(wid * N, N)])


def _run_pass_b(v, m, alpha, src, dst):
    f = pl.kernel(
        _sc_pass_b,
        out_type=(
            jax.ShapeDtypeStruct((NC, N, HID), jnp.float32),
            jax.ShapeDtypeStruct((NW * N,), jnp.float32),
        ),
        mesh=_mesh(),
        scratch_types=[
            pltpu.VMEM((N,), jnp.float32),
            pltpu.VMEM((N,), jnp.float32),
            [pltpu.VMEM((B, HID), jnp.float32)] * 2,
            [pltpu.VMEM((B,), jnp.int32)] * 3,
            [pltpu.VMEM((B,), jnp.int32)] * 3,
            [pltpu.VMEM((B,), jnp.float32)] * 3,
            pltpu.VMEM_SHARED((N, HID), jnp.float32),
            pltpu.SemaphoreType.DMA,
            [pltpu.SemaphoreType.DMA] * 3,
            [pltpu.SemaphoreType.DMA] * 2,
            [pltpu.SemaphoreType.DMA] * 2,
        ],
        compiler_params=_SC_PARAMS,
    )
    return f(v, m, alpha, src, dst)


BN = 2000


def _tc_pre_body(x_ref, w_ref, b_ref, q_ref, k_ref, v_ref, s_ref):
    z = jnp.dot(x_ref[...], w_ref[...],
                preferred_element_type=jnp.float32) + b_ref[...]
    q_ref[...] = z[:, 0 * HID:1 * HID]
    k_ref[...] = z[:, 1 * HID:2 * HID]
    v_ref[...] = z[:, 2 * HID:3 * HID]
    s_ref[...] = z[:, 3 * HID:4 * HID]


def _tc_pre(x, w, b):
    return pl.pallas_call(
        _tc_pre_body,
        grid=(N // BN,),
        in_specs=[
            pl.BlockSpec((BN, HID), lambda i: (i, 0)),
            pl.BlockSpec((HID, 4 * HID), lambda i: (0, 0)),
            pl.BlockSpec((1, 4 * HID), lambda i: (0, 0)),
        ],
        out_specs=[pl.BlockSpec((BN, HID), lambda i: (i, 0))] * 4,
        out_shape=[jax.ShapeDtypeStruct((N, HID), jnp.float32)] * 4,
    )(x, w, b)


def _tc_max_body(ml_ref, m_ref):
    m_ref[...] = jnp.max(ml_ref[...], axis=0, keepdims=True)


def _tc_max(mloc):
    return pl.pallas_call(
        _tc_max_body,
        out_shape=jax.ShapeDtypeStruct((1, N), jnp.float32),
    )(mloc)


def _tc_norm_body(agg_ref, s_ref, h_ref):
    s = jnp.sum(s_ref[...], axis=0)[:, None]
    agg = agg_ref[0] + agg_ref[1]
    h_ref[...] = agg / jnp.maximum(s, 0.5)


def _tc_norm(agg, s2d):
    return pl.pallas_call(
        _tc_norm_body,
        out_shape=jax.ShapeDtypeStruct((N, HID), jnp.float32),
    )(agg, s2d)


def _tc_mid_body(h_ref, skip_ref, cur_ref, w_ref, b_ref,
                 q_ref, k_ref, v_ref, sk_ref, cur_out_ref):
    cur = jnp.maximum(h_ref[...] + skip_ref[...], 0.0) + cur_ref[...]
    cur_out_ref[...] = cur
    z = jnp.dot(cur, w_ref[...], preferred_element_type=jnp.float32) + b_ref[...]
    q_ref[...] = z[:, 0 * HID:1 * HID]
    k_ref[...] = z[:, 1 * HID:2 * HID]
    v_ref[...] = z[:, 2 * HID:3 * HID]
    sk_ref[...] = z[:, 3 * HID:4 * HID]


def _tc_mid(h, skip, cur, w, b):
    return pl.pallas_call(
        _tc_mid_body,
        grid=(N // BN,),
        in_specs=[
            pl.BlockSpec((BN, HID), lambda i: (i, 0)),
            pl.BlockSpec((BN, HID), lambda i: (i, 0)),
            pl.BlockSpec((BN, HID), lambda i: (i, 0)),
            pl.BlockSpec((HID, 4 * HID), lambda i: (0, 0)),
            pl.BlockSpec((1, 4 * HID), lambda i: (0, 0)),
        ],
        out_specs=[pl.BlockSpec((BN, HID), lambda i: (i, 0))] * 5,
        out_shape=[jax.ShapeDtypeStruct((N, HID), jnp.float32)] * 5,
    )(h, skip, cur, w, b)


def _ln(z, g, b):
    mu = jnp.mean(z, axis=-1, keepdims=True)
    var = jnp.mean((z - mu) * (z - mu), axis=-1, keepdims=True)
    return (z - mu) * lax.rsqrt(var + 1e-5) * g + b


def _tc_final_body(h_ref, skip_ref, cur_ref, t3_ref,
                   wt_ref, bt_ref, gt_ref, bt2_ref,
                   wf1_ref, wf2_ref, bf_ref, gf_ref, bf2_ref,
                   wo_ref, bo_ref, o_ref):
    cur = jnp.maximum(h_ref[...] + skip_ref[...], 0.0) + cur_ref[...]
    te = _ln(jnp.maximum(jnp.dot(t3_ref[...], wt_ref[...],
                                 preferred_element_type=jnp.float32)
                         + bt_ref[...], 0.0),
             gt_ref[...], bt2_ref[...])
    comb = (jnp.dot(cur, wf1_ref[...], preferred_element_type=jnp.float32)
            + jnp.dot(te, wf2_ref[...], preferred_element_type=jnp.float32)
            + bf_ref[...])
    fused = _ln(jnp.maximum(comb, 0.0), gf_ref[...], bf2_ref[...])
    o_ref[...] = jnp.dot(fused, wo_ref[...],
                         preferred_element_type=jnp.float32) + bo_ref[...]


def _tc_final(h, skip, cur, t3, wt, bt, gt, bt2, wf1, wf2, bf, gf, bf2,
              wo, bo):
    row = lambda i: (i, 0)
    fixed2 = lambda i: (0, 0)
    return pl.pallas_call(
        _tc_final_body,
        grid=(N // BN,),
        in_specs=[
            pl.BlockSpec((BN, HID), row),
            pl.BlockSpec((BN, HID), row),
            pl.BlockSpec((BN, HID), row),
            pl.BlockSpec((BN, 3), row),
            pl.BlockSpec((3, HID), fixed2),
            pl.BlockSpec((1, HID), fixed2),
            pl.BlockSpec((1, HID), fixed2),
            pl.BlockSpec((1, HID), fixed2),
            pl.BlockSpec((HID, HID), fixed2),
            pl.BlockSpec((HID, HID), fixed2),
            pl.BlockSpec((1, HID), fixed2),
            pl.BlockSpec((1, HID), fixed2),
            pl.BlockSpec((1, HID), fixed2),
            pl.BlockSpec((HID, OUT), fixed2),
            pl.BlockSpec((1, OUT), fixed2),
        ],
        out_specs=pl.BlockSpec((BN, OUT), row),
        out_shape=jax.ShapeDtypeStruct((N, OUT), jnp.float32),
    )(h, skip, cur, t3, wt, bt, gt, bt2, wf1, wf2, bf, gf, bf2, wo, bo)


def kernel(x, edge_index, batch, Wt, bt, gt, bt2, Wsp, bsp, gsp, bsp2,
           Wq, bq, Wk, bk, Wv, bv, Wsk, bsk, Wf, bf, gf, bf2, Wo, bo):
    src = edge_index[0]
    dst = edge_index[1]
    wcat = jnp.concatenate([Wq, Wk, Wv, Wsk], axis=2)
    bcat = jnp.concatenate([bq, bk, bv, bsk], axis=1)[:, None, :]
    t3 = x[:, HID - 3:HID]

    q, k, v, skip = _tc_pre(x, wcat[0], bcat[0])
    cur = x
    for l in range(L):
        alpha, mloc = _run_pass_a(q, k, src, dst)
        m = _tc_max(mloc.reshape(NW, N)).reshape((N,))
        agg, s_flat = _run_pass_b(v, m, alpha, src, dst)
        h = _tc_norm(agg, s_flat.reshape(NW, N))
        if l + 1 < L:
            q, k, v, skip, cur = _tc_mid(h, skip, cur,
                                         wcat[l + 1], bcat[l + 1])
    out = _tc_final(
        h, skip, cur, t3,
        Wt, bt[None, :], gt[None, :], bt2[None, :],
        Wf[:HID], Wf[HID:], bf[None, :], gf[None, :], bf2[None, :],
        Wo, bo[None, :],
    )
    return out


# speedup vs baseline: 1.6201x; 1.6201x over previous
import jax
import jax.numpy as jnp
from jax import lax
from jax.experimental import pallas as pl
from jax.experimental.pallas import tpu as pltpu
from jax.experimental.pallas import tpu_sc as plsc

N = 10000
E = 320000
HID = 128
OUT = 64
L = 8

NC = 2
NS = 16
NW = NC * NS
EPT = E // NW
B = 80
NCHUNK = EPT // B
GROUPS = B // 16
RB = 125
NEG = -1.0e30

_INV_SQRT_HID = 1.0 / (HID ** 0.5)


def _mesh():
    return plsc.VectorSubcoreMesh(
        core_axis_name="c", subcore_axis_name="s", num_cores=NC, num_subcores=NS
    )


_SC_PARAMS = pltpu.CompilerParams(needs_layout_passes=False)
_SC_PARAMS_FLAT = pltpu.CompilerParams(needs_layout_passes=False,
                                       use_tc_tiling_on_sc=False)


def _sc_pass_a(q_hbm, k_hbm, src_hbm, dst_hbm, alpha_out, mloc_out,
               srcs, dsts, alpha_all, mloc, qr0, kr0, qr1, kr1,
               isem, qs0, ks0, qs1, ks1):
    wid = lax.axis_index("s") * NC + lax.axis_index("c")
    base = wid * EPT
    iota = lax.iota(jnp.int32, 16)

    ci1 = pltpu.async_copy(src_hbm.at[pl.ds(base, EPT)], srcs, isem)
    ci2 = pltpu.async_copy(dst_hbm.at[pl.ds(base, EPT)], dsts, isem)

    def init_body(i, _):
        mloc[pl.ds(i * 16, 16)] = jnp.full((16,), NEG, jnp.float32)
        return 0

    lax.fori_loop(0, N // 16, init_body, 0)
    ci1.wait()
    ci2.wait()

    def issue(c, qr, kr, qsem, ksem):
        idx = dsts.at[pl.ds(c * B, B)]
        idxs = srcs.at[pl.ds(c * B, B)]
        cq = pltpu.async_copy(q_hbm.at[idx], qr, qsem)
        ck = pltpu.async_copy(k_hbm.at[idxs], kr, ksem)
        return cq, ck

    def wait(qr, kr, qsem, ksem):
        pltpu.make_async_copy(q_hbm.at[dsts.at[pl.ds(0, B)]], qr, qsem).wait()
        pltpu.make_async_copy(k_hbm.at[srcs.at[pl.ds(0, B)]], kr, ksem).wait()

    def compute(c, qr, kr):
        off = c * B

        def group_body(g, _):
            al = jnp.zeros((16,), jnp.float32)
            for e in range(16):
                row = g * 16 + e
                p0 = jnp.zeros((16,), jnp.float32)
                p1 = jnp.zeros((16,), jnp.float32)
                for c32 in range(HID // 32):
                    qw = plsc.bitcast(qr[row, pl.ds(c32 * 16, 16)], jnp.bfloat16)
                    kw = plsc.bitcast(kr[row, pl.ds(c32 * 16, 16)], jnp.bfloat16)
                    qlo, qhi = plsc.unpack(qw, format=plsc.PackFormat.INTERLEAVED)
                    klo, khi = plsc.unpack(kw, format=plsc.PackFormat.INTERLEAVED)
                    p0 = p0 + qlo * klo
                    p1 = p1 + qhi * khi
                b = p0 + p1
                for st in (1, 2, 4, 8):
                    b = b + b.at[iota ^ st].get(mode="promise_in_bounds")
                al = jnp.where(iota == e, b, al)
            al = al * _INV_SQRT_HID
            alpha_all[pl.ds(off + g * 16, 16)] = al
            dstv = dsts[pl.ds(off + g * 16, 16)]
            ka, da = plsc.sort_key_val(al, dstv)
            old = plsc.load_gather(mloc, [da])
            plsc.store_scatter(mloc, [da], jnp.maximum(old, ka))
            kd, dd = plsc.sort_key_val(al, dstv, descending=True)
            old2 = plsc.load_gather(mloc, [dd])
            plsc.store_scatter(mloc, [dd], jnp.maximum(old2, kd))
            return 0

        lax.fori_loop(0, GROUPS, group_body, 0)

    i0 = issue(0, qr0, kr0, qs0, ks0)

    def pair_body(tt, _):
        c0 = 2 * tt
        c1 = c0 + 1

        @pl.when(c1 < NCHUNK)
        def _():
            issue(c1, qr1, kr1, qs1, ks1)

        wait(qr0, kr0, qs0, ks0)
        compute(c0, qr0, kr0)

        @pl.when(c1 < NCHUNK)
        def _():
            @pl.when(c1 + 1 < NCHUNK)
            def _():
                issue(c1 + 1, qr0, kr0, qs0, ks0)

            wait(qr1, kr1, qs1, ks1)
            compute(c1, qr1, kr1)

        return 0

    lax.fori_loop(0, (NCHUNK + 1) // 2, pair_body, 0)

    pltpu.sync_copy(alpha_all, alpha_out.at[pl.ds(base, EPT)])
    pltpu.sync_copy(mloc, mloc_out.at[pl.ds(wid * N, N)])


def _run_pass_a(q, k, src, dst):
    f = pl.kernel(
        _sc_pass_a,
        out_type=(
            jax.ShapeDtypeStruct((E,), jnp.float32),
            jax.ShapeDtypeStruct((NW * N,), jnp.float32),
        ),
        mesh=_mesh(),
        scratch_types=[
            pltpu.VMEM((EPT,), jnp.int32),
            pltpu.VMEM((EPT,), jnp.int32),
            pltpu.VMEM((EPT,), jnp.float32),
            pltpu.VMEM((N,), jnp.float32),
            pltpu.VMEM((B, HID // 2), jnp.float32),
            pltpu.VMEM((B, HID // 2), jnp.float32),
            pltpu.VMEM((B, HID // 2), jnp.float32),
            pltpu.VMEM((B, HID // 2), jnp.float32),
            pltpu.SemaphoreType.DMA,
            pltpu.SemaphoreType.DMA,
            pltpu.SemaphoreType.DMA,
            pltpu.SemaphoreType.DMA,
            pltpu.SemaphoreType.DMA,
        ],
        compiler_params=_SC_PARAMS_FLAT,
    )
    return f(q, k, src, dst)


def _sc_pass_b(v_hbm, m_hbm, alpha_hbm, src_hbm, dst_hbm, agg_out, s_out,
               m_vm, s_loc, vrs, idxs, idxd, albs, aggsp,
               msem, isems, vsems, ssems):
    cid = lax.axis_index("c")
    sid = lax.axis_index("s")
    wid = sid * NC + cid
    base = wid * EPT
    iota = lax.iota(jnp.int32, 16)

    cm = pltpu.async_copy(m_hbm, m_vm, msem)

    def zinit_body(i, _):
        s_loc[pl.ds(i * 16, 16)] = jnp.zeros((16,), jnp.float32)
        return 0

    lax.fori_loop(0, N // 16, zinit_body, 0)

    def zrow_body(r, _):
        for cc in range(HID // 16):
            vrs[0][r, pl.ds(cc * 16, 16)] = jnp.zeros((16,), jnp.float32)
        return 0

    lax.fori_loop(0, B, zrow_body, 0)

    for t in range(RB // NS + 1):
        bi = t * NS + sid

        @pl.when(bi < RB)
        def _():
            off = pl.multiple_of(bi * B, B)
            pltpu.sync_copy(vrs[0], aggsp.at[pl.ds(off, B)])

    cm.wait()
    plsc.subcore_barrier()

    def issue_idx(c, j):
        off = c * B
        pltpu.async_copy(src_hbm.at[pl.ds(off + base, B)], idxs[j], isems[j])
        pltpu.async_copy(dst_hbm.at[pl.ds(off + base, B)], idxd[j], isems[j])
        pltpu.async_copy(alpha_hbm.at[pl.ds(off + base, B)], albs[j], isems[j])

    def wait_idx(j):
        pltpu.make_async_copy(src_hbm.at[pl.ds(0, B)], idxs[j], isems[j]).wait()
        pltpu.make_async_copy(dst_hbm.at[pl.ds(0, B)], idxd[j], isems[j]).wait()
        pltpu.make_async_copy(alpha_hbm.at[pl.ds(0, B)], albs[j], isems[j]).wait()

    def issue_v(j, p):
        pltpu.async_copy(v_hbm.at[idxs[j]], vrs[p], vsems[p])

    def wait_v(p):
        pltpu.make_async_copy(v_hbm.at[pl.ds(0, B)], vrs[p], vsems[p]).wait()

    def issue_scat(j, p):
        pltpu.async_copy(vrs[p], aggsp.at[idxd[j]], ssems[p], add=True)

    def wait_scat(p):
        pltpu.make_async_copy(vrs[p], aggsp.at[pl.ds(0, B)], ssems[p]).wait()

    def compute(j, p):
        vr = vrs[p]

        def group_body(g, _):
            al = albs[j][pl.ds(g * 16, 16)]
            dstv = idxd[j][pl.ds(g * 16, 16)]
            mg = plsc.load_gather(m_vm, [dstv])
            t = jnp.exp(al - mg)
            plsc.addupdate_scatter(s_loc, [dstv], t)
            for e in range(16):
                row = g * 16 + e
                te = t.at[jnp.full((16,), e, jnp.int32)].get(mode="promise_in_bounds")
                for c8 in range(HID // 16):
                    sl = pl.ds(c8 * 16, 16)
                    vr[row, sl] = vr[row, sl] * te
            return 0

        lax.fori_loop(0, GROUPS, group_body, 0)

    issue_idx(0, 0)
    issue_idx(1, 1)
    issue_idx(2, 2)
    wait_idx(0)
    issue_v(0, 0)

    def chunk_step(c, u):
        j = u % 3
        p = u % 2
        j1 = (u + 1) % 3
        j2 = (u + 2) % 3
        p1 = (u + 1) % 2

        @pl.when(c + 1 < NCHUNK)
        def _():
            wait_idx(j1)

        @pl.when(c >= 1)
        def _():
            wait_scat(p1)

        @pl.when(c + 1 < NCHUNK)
        def _():
            issue_v(j1, p1)

        @pl.when((c + 2 < NCHUNK) & (c >= 1))
        def _():
            issue_idx(c + 2, j2)

        wait_v(p)
        compute(j, p)
        issue_scat(j, p)

    def hex_body(h, _):
        for u in range(6):
            c = 6 * h + u

            @pl.when(c < NCHUNK)
            def _():
                chunk_step(c, u)

        return 0

    lax.fori_loop(0, (NCHUNK + 5) // 6, hex_body, 0)

    wait_scat((NCHUNK - 1) % 2)
    plsc.subcore_barrier()

    for t in range(RB // NS + 1):
        bi = t * NS + sid

        @pl.when(bi < RB)
        def _():
            off = pl.multiple_of(bi * B, B)
            pltpu.sync_copy(aggsp.at[pl.ds(off, B)],
                            agg_out.at[cid, pl.ds(off, B)])

    pltpu.sync_copy(s_loc, s_out.at[pl.ds(wid * N, N)])


def _run_pass_b(v, m, alpha, src, dst):
    f = pl.kernel(
        _sc_pass_b,
        out_type=(
            jax.ShapeDtypeStruct((NC, N, HID), jnp.float32),
            jax.ShapeDtypeStruct((NW * N,), jnp.float32),
        ),
        mesh=_mesh(),
        scratch_types=[
            pltpu.VMEM((N,), jnp.float32),
            pltpu.VMEM((N,), jnp.float32),
            [pltpu.VMEM((B, HID), jnp.float32)] * 2,
            [pltpu.VMEM((B,), jnp.int32)] * 3,
            [pltpu.VMEM((B,), jnp.int32)] * 3,
            [pltpu.VMEM((B,), jnp.float32)] * 3,
            pltpu.VMEM_SHARED((N, HID), jnp.float32),
            pltpu.SemaphoreType.DMA,
            [pltpu.SemaphoreType.DMA] * 3,
            [pltpu.SemaphoreType.DMA] * 2,
            [pltpu.SemaphoreType.DMA] * 2,
        ],
        compiler_params=_SC_PARAMS,
    )
    return f(v, m, alpha, src, dst)


BN = 2000


def _tc_pre_body(x_ref, w_ref, b_ref, q_ref, k_ref, v_ref, s_ref):
    z = jnp.dot(x_ref[...], w_ref[...],
                preferred_element_type=jnp.float32) + b_ref[...]
    q_ref[...] = z[:, 0 * HID:1 * HID].astype(jnp.bfloat16)
    k_ref[...] = z[:, 1 * HID:2 * HID].astype(jnp.bfloat16)
    v_ref[...] = z[:, 2 * HID:3 * HID]
    s_ref[...] = z[:, 3 * HID:4 * HID]


def _tc_pre(x, w, b):
    return pl.pallas_call(
        _tc_pre_body,
        grid=(N // BN,),
        in_specs=[
            pl.BlockSpec((BN, HID), lambda i: (i, 0)),
            pl.BlockSpec((HID, 4 * HID), lambda i: (0, 0)),
            pl.BlockSpec((1, 4 * HID), lambda i: (0, 0)),
        ],
        out_specs=[pl.BlockSpec((BN, HID), lambda i: (i, 0))] * 4,
        out_shape=[jax.ShapeDtypeStruct((N, HID), jnp.bfloat16)] * 2
        + [jax.ShapeDtypeStruct((N, HID), jnp.float32)] * 2,
    )(x, w, b)


def _tc_max_body(ml_ref, m_ref):
    m_ref[...] = jnp.max(ml_ref[...], axis=0, keepdims=True)


def _tc_max(mloc):
    return pl.pallas_call(
        _tc_max_body,
        out_shape=jax.ShapeDtypeStruct((1, N), jnp.float32),
    )(mloc)


def _tc_norm_body(agg_ref, s_ref, h_ref):
    s = jnp.sum(s_ref[...], axis=0)[:, None]
    agg = agg_ref[0] + agg_ref[1]
    h_ref[...] = agg / jnp.maximum(s, 0.5)


def _tc_norm(agg, s2d):
    return pl.pallas_call(
        _tc_norm_body,
        out_shape=jax.ShapeDtypeStruct((N, HID), jnp.float32),
    )(agg, s2d)


def _tc_mid_body(h_ref, skip_ref, cur_ref, w_ref, b_ref,
                 q_ref, k_ref, v_ref, sk_ref, cur_out_ref):
    cur = jnp.maximum(h_ref[...] + skip_ref[...], 0.0) + cur_ref[...]
    cur_out_ref[...] = cur
    z = jnp.dot(cur, w_ref[...], preferred_element_type=jnp.float32) + b_ref[...]
    q_ref[...] = z[:, 0 * HID:1 * HID].astype(jnp.bfloat16)
    k_ref[...] = z[:, 1 * HID:2 * HID].astype(jnp.bfloat16)
    v_ref[...] = z[:, 2 * HID:3 * HID]
    sk_ref[...] = z[:, 3 * HID:4 * HID]


def _tc_mid(h, skip, cur, w, b):
    return pl.pallas_call(
        _tc_mid_body,
        grid=(N // BN,),
        in_specs=[
            pl.BlockSpec((BN, HID), lambda i: (i, 0)),
            pl.BlockSpec((BN, HID), lambda i: (i, 0)),
            pl.BlockSpec((BN, HID), lambda i: (i, 0)),
            pl.BlockSpec((HID, 4 * HID), lambda i: (0, 0)),
            pl.BlockSpec((1, 4 * HID), lambda i: (0, 0)),
        ],
        out_specs=[pl.BlockSpec((BN, HID), lambda i: (i, 0))] * 5,
        out_shape=[jax.ShapeDtypeStruct((N, HID), jnp.bfloat16)] * 2
        + [jax.ShapeDtypeStruct((N, HID), jnp.float32)] * 3,
    )(h, skip, cur, w, b)


def _ln(z, g, b):
    mu = jnp.mean(z, axis=-1, keepdims=True)
    var = jnp.mean((z - mu) * (z - mu), axis=-1, keepdims=True)
    return (z - mu) * lax.rsqrt(var + 1e-5) * g + b


def _tc_final_body(h_ref, skip_ref, cur_ref, t3_ref,
                   wt_ref, bt_ref, gt_ref, bt2_ref,
                   wf1_ref, wf2_ref, bf_ref, gf_ref, bf2_ref,
                   wo_ref, bo_ref, o_ref):
    cur = jnp.maximum(h_ref[...] + skip_ref[...], 0.0) + cur_ref[...]
    te = _ln(jnp.maximum(jnp.dot(t3_ref[...], wt_ref[...],
                                 preferred_element_type=jnp.float32)
                         + bt_ref[...], 0.0),
             gt_ref[...], bt2_ref[...])
    comb = (jnp.dot(cur, wf1_ref[...], preferred_element_type=jnp.float32)
            + jnp.dot(te, wf2_ref[...], preferred_element_type=jnp.float32)
            + bf_ref[...])
    fused = _ln(jnp.maximum(comb, 0.0), gf_ref[...], bf2_ref[...])
    o_ref[...] = jnp.dot(fused, wo_ref[...],
                         preferred_element_type=jnp.float32) + bo_ref[...]


def _tc_final(h, skip, cur, t3, wt, bt, gt, bt2, wf1, wf2, bf, gf, bf2,
              wo, bo):
    row = lambda i: (i, 0)
    fixed2 = lambda i: (0, 0)
    return pl.pallas_call(
        _tc_final_body,
        grid=(N // BN,),
        in_specs=[
            pl.BlockSpec((BN, HID), row),
            pl.BlockSpec((BN, HID), row),
            pl.BlockSpec((BN, HID), row),
            pl.BlockSpec((BN, 3), row),
            pl.BlockSpec((3, HID), fixed2),
            pl.BlockSpec((1, HID), fixed2),
            pl.BlockSpec((1, HID), fixed2),
            pl.BlockSpec((1, HID), fixed2),
            pl.BlockSpec((HID, HID), fixed2),
            pl.BlockSpec((HID, HID), fixed2),
            pl.BlockSpec((1, HID), fixed2),
            pl.BlockSpec((1, HID), fixed2),
            pl.BlockSpec((1, HID), fixed2),
            pl.BlockSpec((HID, OUT), fixed2),
            pl.BlockSpec((1, OUT), fixed2),
        ],
        out_specs=pl.BlockSpec((BN, OUT), row),
        out_shape=jax.ShapeDtypeStruct((N, OUT), jnp.float32),
    )(h, skip, cur, t3, wt, bt, gt, bt2, wf1, wf2, bf, gf, bf2, wo, bo)


def kernel(x, edge_index, batch, Wt, bt, gt, bt2, Wsp, bsp, gsp, bsp2,
           Wq, bq, Wk, bk, Wv, bv, Wsk, bsk, Wf, bf, gf, bf2, Wo, bo):
    src = edge_index[0]
    dst = edge_index[1]
    wcat = jnp.concatenate([Wq, Wk, Wv, Wsk], axis=2)
    bcat = jnp.concatenate([bq, bk, bv, bsk], axis=1)[:, None, :]
    t3 = x[:, HID - 3:HID]

    def pack32(t):
        return lax.bitcast_convert_type(t.reshape(N, HID // 2, 2), jnp.float32)

    q, k, v, skip = _tc_pre(x, wcat[0], bcat[0])
    cur = x
    for l in range(L):
        alpha, mloc = _run_pass_a(pack32(q), pack32(k), src, dst)
        m = _tc_max(mloc.reshape(NW, N)).reshape((N,))
        agg, s_flat = _run_pass_b(v, m, alpha, src, dst)
        h = _tc_norm(agg, s_flat.reshape(NW, N))
        if l + 1 < L:
            q, k, v, skip, cur = _tc_mid(h, skip, cur,
                                         wcat[l + 1], bcat[l + 1])
    out = _tc_final(
        h, skip, cur, t3,
        Wt, bt[None, :], gt[None, :], bt2[None, :],
        Wf[:HID], Wf[HID:], bf[None, :], gf[None, :], bf2[None, :],
        Wo, bo[None, :],
    )
    return out
